# Initial kernel scaffold; baseline (speedup 1.0000x reference)
import functools

import jax
import jax.numpy as jnp
from jax import lax
from jax.experimental import pallas as pl
from jax.experimental.pallas import tpu as pltpu
from jax.experimental.pallas import tpu_sc as plsc

N = 10000
D = 128
E = 320000
NL = 5
EPS = 1e-5

NC, NS = 2, 16
NW = NC * NS
CH = 128
NCH = 80
EPT = CH * NCH
E_PAD = EPT * NW
ROWS_Z = 628
NACC = ROWS_Z * NS
ROWS_O = N // NS


def _make_agg():
    mesh = plsc.VectorSubcoreMesh(core_axis_name="c", subcore_axis_name="s")

    @functools.partial(
        pl.kernel,
        out_type=jax.ShapeDtypeStruct((NC, N, D), jnp.float32),
        mesh=mesh,
        scratch_types=[
            pltpu.VMEM((CH,), jnp.int32),
            pltpu.VMEM((CH,), jnp.int32),
            pltpu.VMEM((CH, D), jnp.float32),
            pltpu.VMEM_SHARED((NACC, D), jnp.float32),
            pltpu.SemaphoreType.DMA,
        ],
    )
    def agg(h_hbm, src_hbm, dst_hbm, z_hbm, out_hbm, sidx, didx, rows, acc, sem):
        c = lax.axis_index("c")
        s = lax.axis_index("s")
        wid = c * NS + s
        pltpu.sync_copy(z_hbm.at[pl.ds(s * ROWS_Z, ROWS_Z)],
                        acc.at[pl.ds(s * ROWS_Z, ROWS_Z)])
        plsc.subcore_barrier()

        @pl.loop(0, NCH)
        def _(i):
            base = wid * EPT + i * CH
            pltpu.sync_copy(src_hbm.at[pl.ds(base, CH)], sidx)
            pltpu.sync_copy(dst_hbm.at[pl.ds(base, CH)], didx)
            pltpu.async_copy(h_hbm.at[sidx], rows, sem).wait()
            pltpu.sync_copy(rows, acc.at[didx], add=True)

        plsc.subcore_barrier()
        pltpu.sync_copy(acc.at[pl.ds(s * ROWS_O, ROWS_O)],
                        out_hbm.at[c, pl.ds(s * ROWS_O, ROWS_O)])

    return agg


_agg = _make_agg()


def _mlp_body(x_ref, agg_ref, w1_ref, b1_ref, g_ref, be_ref, w2_ref, b2_ref,
              o_ref):
    h = x_ref[...] + agg_ref[0] + agg_ref[1]
    t = jnp.dot(h, w1_ref[...], preferred_element_type=jnp.float32) + b1_ref[...]
    m = jnp.mean(t, axis=0, keepdims=True)
    v = jnp.mean((t - m) * (t - m), axis=0, keepdims=True)
    tn = (t - m) * lax.rsqrt(v + EPS) * g_ref[...] + be_ref[...]
    r = jnp.maximum(tn, 0.0)
    o_ref[...] = jnp.maximum(
        jnp.dot(r, w2_ref[...], preferred_element_type=jnp.float32)
        + b2_ref[...], 0.0)


_mlp = pl.pallas_call(
    _mlp_body,
    out_shape=jax.ShapeDtypeStruct((N, D), jnp.float32),
)


def kernel(x, edge_index, W1s, b1s, gammas, betas, W2s, b2s):
    src = edge_index[0]
    dst = edge_index[1]
    pad = E_PAD - E
    src_p = jnp.concatenate([src, jnp.zeros((pad,), jnp.int32)])
    dst_p = jnp.concatenate([dst, jnp.full((pad,), N, jnp.int32)])
    zeros = jnp.zeros((NACC, D), jnp.float32)
    h = x
    for l in range(NL):
        parts = _agg(h, src_p, dst_p, zeros)
        h = _mlp(h, parts, W1s[l], b1s[l].reshape(1, D),
                 gammas[l].reshape(1, D), betas[l].reshape(1, D),
                 W2s[l], b2s[l].reshape(1, D))
    return h


# speedup vs baseline: 2.4398x; 2.4398x over previous
import functools

import jax
import jax.numpy as jnp
from jax import lax
from jax.experimental import pallas as pl
from jax.experimental.pallas import tpu as pltpu
from jax.experimental.pallas import tpu_sc as plsc

N = 10000
D = 128
E = 320000
NL = 5
EPS = 1e-5

NC, NS = 2, 16
NW = NC * NS
CH = 128
NCH = 80
EPT = CH * NCH
ROWS_Z = 632
NACC = ROWS_Z * NS
NB = N // 8


def _make_agg():
    mesh = plsc.VectorSubcoreMesh(core_axis_name="c", subcore_axis_name="s")

    @functools.partial(
        pl.kernel,
        out_type=jax.ShapeDtypeStruct((NC, NACC, D), jnp.float32),
        mesh=mesh,
        scratch_types=[
            pltpu.VMEM((CH,), jnp.int32),
            pltpu.VMEM((CH,), jnp.int32),
            pltpu.VMEM((CH, D), jnp.float32),
            pltpu.VMEM_SHARED((NACC, D), jnp.float32),
            pltpu.SemaphoreType.DMA,
        ],
    )
    def agg(h_hbm, src_hbm, dst_hbm, z_hbm, out_hbm, sidx, didx, rows, acc, sem):
        c = lax.axis_index("c")
        s = lax.axis_index("s")
        wid = c * NS + s
        pltpu.sync_copy(z_hbm.at[pl.ds(s * ROWS_Z, ROWS_Z)],
                        acc.at[pl.ds(s * ROWS_Z, ROWS_Z)])
        plsc.subcore_barrier()

        @pl.loop(0, NCH)
        def _(i):
            base = wid * EPT + i * CH
            pltpu.sync_copy(src_hbm.at[pl.ds(base, CH)], sidx)
            pltpu.sync_copy(dst_hbm.at[pl.ds(base, CH)], didx)
            pltpu.async_copy(h_hbm.at[sidx], rows, sem).wait()
            pltpu.sync_copy(rows, acc.at[didx], add=True)

        plsc.subcore_barrier()
        pltpu.sync_copy(acc.at[pl.ds(s * ROWS_Z, ROWS_Z)],
                        out_hbm.at[c, pl.ds(s * ROWS_Z, ROWS_Z)])

    return agg


_agg = _make_agg()


def _mlp_body(h_ref, p0_ref, p1_ref, w1_ref, b1_ref, g_ref, be_ref, w2_ref,
              b2_ref, o_ref, t_ref):
    hh = h_ref[...] + p0_ref[...] + p1_ref[...]
    t = jnp.dot(hh, w1_ref[...], preferred_element_type=jnp.float32) + b1_ref[...]
    t_ref[...] = t.reshape(NB, 8, D)

    def halve(a):
        a = a[:4] + a[4:]
        a = a[:2] + a[2:]
        return a[0:1] + a[1:2]

    m = halve(lax.fori_loop(
        0, NB, lambda i, a: a + t_ref[i], jnp.zeros((8, D), jnp.float32)
    )) * jnp.float32(1e-4)

    def vchunk(lo, hi):
        def step(i, a):
            d = t_ref[i] - m
            return a + d * d
        return halve(lax.fori_loop(lo, hi, step, jnp.zeros((8, D), jnp.float32)))

    v = (vchunk(0, NB // 2) + vchunk(NB // 2, NB)) * jnp.float32(1e-4)

    tn = (t - m) / jnp.sqrt(v + EPS) * g_ref[...] + be_ref[...]
    r = jnp.maximum(tn, 0.0)
    o_ref[...] = jnp.maximum(
        jnp.dot(r, w2_ref[...], preferred_element_type=jnp.float32)
        + b2_ref[...], 0.0)


_mlp = pl.pallas_call(
    _mlp_body,
    out_shape=jax.ShapeDtypeStruct((N, D), jnp.float32),
    scratch_shapes=[pltpu.VMEM((NB, 8, D), jnp.float32)],
)


def kernel(x, edge_index, W1s, b1s, gammas, betas, W2s, b2s):
    src = edge_index[0]
    dst = edge_index[1]
    order = jnp.argsort(dst, stable=True)
    src_s = src[order]
    dst_s = dst[order]
    cuts = (jnp.arange(1, NW) * (E // NW)).astype(jnp.int32)
    starts = jnp.concatenate([
        jnp.zeros((1,), jnp.int32),
        jnp.searchsorted(dst_s, dst_s[cuts], side="left").astype(jnp.int32),
    ])
    ends = jnp.concatenate([starts[1:], jnp.full((1,), E, jnp.int32)])
    idx = starts[:, None] + jnp.arange(EPT, dtype=jnp.int32)[None, :]
    valid = idx < ends[:, None]
    idxc = jnp.minimum(idx, E - 1)
    src_p = jnp.where(valid, src_s[idxc], 0).reshape(-1)
    dst_p = jnp.where(valid, dst_s[idxc], N).reshape(-1)
    zeros = jnp.zeros((NACC, D), jnp.float32)

    h = x
    for l in range(NL):
        parts = _agg(h, src_p, dst_p, zeros)
        h = _mlp(h, parts[0, :N], parts[1, :N], W1s[l],
                 b1s[l].reshape(1, D), gammas[l].reshape(1, D),
                 betas[l].reshape(1, D), W2s[l], b2s[l].reshape(1, D))
    return h
